# Initial kernel scaffold; baseline (speedup 1.0000x reference)
import jax
import jax.numpy as jnp
from jax.experimental import pallas as pl


def kernel(x, edge_index, batch, W1, b1, W2, b2):
    raise NotImplementedError("write your pallas kernel here")


# speedup vs baseline: 160.2541x; 160.2541x over previous
import functools

import jax
import jax.numpy as jnp
from jax import lax
from jax.experimental import pallas as pl
from jax.experimental.pallas import tpu as pltpu
from jax.experimental.pallas import tpu_sc as plsc

N_NODES = 100000
N_EDGES = 3200000
NUM_GRAPHS = 64

NC = 2
NS = 16
L = 16
NW = NC * NS

NP = 102400
PT_NODES = NP // NW
PSC_NODES = NP // NS
ET = N_EDGES // NW
C = 4000
NCHUNK = ET // C

PAD_GRAPH = 512
ACC_BINS = 1024

_MESH = plsc.VectorSubcoreMesh(core_axis_name="c", subcore_axis_name="s")
_SC_PARAMS = pltpu.CompilerParams(needs_layout_passes=False)


def _fill(ref, n, val):
    v = jnp.full((L,), val, ref.dtype)

    def body(j, carry):
        ref[pl.ds(j * L, L)] = v
        return carry

    lax.fori_loop(0, n // L, body, 0)


def _zero_shared_slice(acc, zbuf, zlen, sid, per_tile):
    base = sid * per_tile
    off = 0
    while off < per_tile:
        n = min(zlen, per_tile - off)
        pltpu.sync_copy(zbuf.at[pl.ds(0, n)], acc.at[pl.ds(base + off, n)])
        off += n


@functools.partial(
    pl.kernel,
    out_type=jax.ShapeDtypeStruct((NC, NP), jnp.float32),
    mesh=_MESH,
    compiler_params=_SC_PARAMS,
    scratch_types=[
        pltpu.VMEM((C,), jnp.int32),
        pltpu.VMEM((C,), jnp.float32),
        pltpu.VMEM_SHARED((NP,), jnp.float32),
    ],
)
def _deg_kernel(dst_hbm, out_hbm, dst_v, ones_v, acc):
    cid = lax.axis_index("c")
    sid = lax.axis_index("s")
    wid = sid * NC + cid
    _fill(ones_v, C, 0.0)
    _zero_shared_slice(acc, ones_v, C, sid, PSC_NODES)
    _fill(ones_v, C, 1.0)
    plsc.subcore_barrier()
    ebase = wid * ET

    def chunk(ci, carry):
        pltpu.sync_copy(dst_hbm.at[pl.ds(ebase + ci * C, C)], dst_v)
        pltpu.sync_copy(ones_v, acc.at[dst_v], add=True)
        return carry

    lax.fori_loop(0, NCHUNK, chunk, 0)
    plsc.subcore_barrier()
    sl = pl.ds(sid * PSC_NODES, PSC_NODES)
    pltpu.sync_copy(acc.at[sl], out_hbm.at[cid, sl])


def _dinv_y_tc_body(degp_ref, x_ref, dinv_ref, y_ref):
    deg = degp_ref[0] + degp_ref[1] + 1.0
    dinv = lax.rsqrt(deg)
    dinv_ref[...] = dinv
    y_ref[...] = x_ref[...] * dinv


_dinv_y_tc = pl.pallas_call(
    _dinv_y_tc_body,
    out_shape=(
        jax.ShapeDtypeStruct((NP // 128, 128), jnp.float32),
        jax.ShapeDtypeStruct((NP // 128, 128), jnp.float32),
    ),
)


@functools.partial(
    pl.kernel,
    out_type=jax.ShapeDtypeStruct((NC, NP), jnp.float32),
    mesh=_MESH,
    compiler_params=_SC_PARAMS,
    scratch_types=[
        pltpu.VMEM((NP,), jnp.float32),
        pltpu.VMEM((C,), jnp.int32),
        pltpu.VMEM((C,), jnp.int32),
        pltpu.VMEM((C,), jnp.float32),
        pltpu.VMEM_SHARED((NP,), jnp.float32),
    ],
)
def _edge_agg_kernel(src_hbm, dst_hbm, tab_hbm, out_hbm,
                     tab_v, src_v, dst_v, vals_v, acc):
    cid = lax.axis_index("c")
    sid = lax.axis_index("s")
    wid = sid * NC + cid
    pltpu.sync_copy(tab_hbm, tab_v)
    _fill(vals_v, C, 0.0)
    _zero_shared_slice(acc, vals_v, C, sid, PSC_NODES)
    plsc.subcore_barrier()
    ebase = wid * ET

    def chunk(ci, carry):
        pltpu.sync_copy(src_hbm.at[pl.ds(ebase + ci * C, C)], src_v)
        pltpu.sync_copy(dst_hbm.at[pl.ds(ebase + ci * C, C)], dst_v)

        def g(j, c2):
            s = pl.ds(j * L, L)
            vals_v[s] = plsc.load_gather(tab_v, [src_v[s]])
            return c2

        lax.fori_loop(0, C // L, g, 0)
        pltpu.sync_copy(vals_v, acc.at[dst_v], add=True)
        return carry

    lax.fori_loop(0, NCHUNK, chunk, 0)
    plsc.subcore_barrier()
    sl = pl.ds(sid * PSC_NODES, PSC_NODES)
    pltpu.sync_copy(acc.at[sl], out_hbm.at[cid, sl])


@functools.partial(
    pl.kernel,
    out_type=(
        jax.ShapeDtypeStruct((NP,), jnp.float32),
        jax.ShapeDtypeStruct((NP,), jnp.float32),
    ),
    mesh=_MESH,
    compiler_params=_SC_PARAMS,
    scratch_types=[
        pltpu.VMEM((PT_NODES,), jnp.float32),
        pltpu.VMEM((PT_NODES,), jnp.float32),
        pltpu.VMEM((PT_NODES,), jnp.float32),
        pltpu.VMEM((PT_NODES,), jnp.float32),
        pltpu.VMEM((PT_NODES,), jnp.float32),
        pltpu.VMEM((PT_NODES,), jnp.float32),
        pltpu.VMEM((16, L), jnp.float32),
        pltpu.VMEM((16, L), jnp.float32),
        pltpu.VMEM((16, L), jnp.float32),
        pltpu.VMEM((16, L), jnp.float32),
    ],
)
def _feat_kernel(aggp_hbm, dinv_hbm, y_hbm, w1_hbm, b1_hbm, w20_hbm, w21_hbm,
                 z0_hbm, z1_hbm,
                 p0_v, p1_v, dinv_v, y_v, z0_v, z1_v, w1_v, b1_v, w20_v, w21_v):
    cid = lax.axis_index("c")
    sid = lax.axis_index("s")
    wid = sid * NC + cid
    base = wid * PT_NODES
    sl = pl.ds(base, PT_NODES)
    pltpu.sync_copy(aggp_hbm.at[0, sl], p0_v)
    pltpu.sync_copy(aggp_hbm.at[1, sl], p1_v)
    pltpu.sync_copy(dinv_hbm.at[sl], dinv_v)
    pltpu.sync_copy(y_hbm.at[sl], y_v)
    pltpu.sync_copy(w1_hbm, w1_v)
    pltpu.sync_copy(b1_hbm, b1_v)
    pltpu.sync_copy(w20_hbm, w20_v)
    pltpu.sync_copy(w21_hbm, w21_v)

    zero = jnp.zeros((L,), jnp.float32)

    def body(j, carry):
        s = pl.ds(j * L, L)
        dv = dinv_v[s]
        s1 = dv * (p0_v[s] + p1_v[s] + y_v[s])
        g0 = zero
        g1 = zero
        for f in range(16):
            h = jnp.maximum(s1 * w1_v[f, :] + b1_v[f, :], 0.0)
            g0 = g0 + h * w20_v[f, :]
            g1 = g1 + h * w21_v[f, :]
        z0_v[s] = g0 * dv
        z1_v[s] = g1 * dv
        return carry

    lax.fori_loop(0, PT_NODES // L, body, 0)
    pltpu.sync_copy(z0_v, z0_hbm.at[sl])
    pltpu.sync_copy(z1_v, z1_hbm.at[sl])


@functools.partial(
    pl.kernel,
    out_type=jax.ShapeDtypeStruct((NC, ACC_BINS), jnp.float32),
    mesh=_MESH,
    compiler_params=_SC_PARAMS,
    scratch_types=[
        pltpu.VMEM((PT_NODES,), jnp.float32),
        pltpu.VMEM((PT_NODES,), jnp.float32),
        pltpu.VMEM((PT_NODES,), jnp.float32),
        pltpu.VMEM((PT_NODES,), jnp.float32),
        pltpu.VMEM((PT_NODES,), jnp.float32),
        pltpu.VMEM((PT_NODES,), jnp.float32),
        pltpu.VMEM((PT_NODES,), jnp.float32),
        pltpu.VMEM((PT_NODES,), jnp.int32),
        pltpu.VMEM((PT_NODES,), jnp.float32),
        pltpu.VMEM((PT_NODES,), jnp.float32),
        pltpu.VMEM((PT_NODES,), jnp.int32),
        pltpu.VMEM((PT_NODES,), jnp.int32),
        pltpu.VMEM((2, L), jnp.float32),
        pltpu.VMEM((ACC_BINS // NS,), jnp.float32),
        pltpu.VMEM_SHARED((ACC_BINS,), jnp.float32),
    ],
)
def _pool_kernel(a0p_hbm, a1p_hbm, z0_hbm, z1_hbm, dinv_hbm, batch_hbm, b2_hbm,
                 out_hbm,
                 q00_v, q01_v, q10_v, q11_v, z0_v, z1_v, dinv_v, batch_v,
                 h0_v, h1_v, ib_v, ic_v, b2_v, zb_v, acc):
    cid = lax.axis_index("c")
    sid = lax.axis_index("s")
    wid = sid * NC + cid
    base = wid * PT_NODES
    sl = pl.ds(base, PT_NODES)
    pltpu.sync_copy(a0p_hbm.at[0, sl], q00_v)
    pltpu.sync_copy(a0p_hbm.at[1, sl], q01_v)
    pltpu.sync_copy(a1p_hbm.at[0, sl], q10_v)
    pltpu.sync_copy(a1p_hbm.at[1, sl], q11_v)
    pltpu.sync_copy(z0_hbm.at[sl], z0_v)
    pltpu.sync_copy(z1_hbm.at[sl], z1_v)
    pltpu.sync_copy(dinv_hbm.at[sl], dinv_v)
    pltpu.sync_copy(batch_hbm.at[sl], batch_v)
    pltpu.sync_copy(b2_hbm, b2_v)

    _fill(zb_v, ACC_BINS // NS, 0.0)
    _zero_shared_slice(acc, zb_v, ACC_BINS // NS, sid, ACC_BINS // NS)

    k64 = jnp.full((L,), 64, jnp.int32)
    k256 = jnp.full((L,), 256, jnp.int32)

    def body(j, carry):
        s = pl.ds(j * L, L)
        dv = dinv_v[s]
        p = dv * (q00_v[s] + q01_v[s] + z0_v[s])
        q = dv * (q10_v[s] + q11_v[s] + z1_v[s])
        h0_v[s] = jnp.maximum(p + b2_v[0, :], 0.0)
        h1_v[s] = jnp.maximum(q + b2_v[1, :], 0.0)
        bi = batch_v[s]
        ib_v[s] = bi + k64
        ic_v[s] = bi + k256
        return carry

    lax.fori_loop(0, PT_NODES // L, body, 0)
    plsc.subcore_barrier()
    pltpu.sync_copy(h0_v, acc.at[batch_v], add=True)
    pltpu.sync_copy(h1_v, acc.at[ib_v], add=True)

    def body2(j, carry):
        h0_v[pl.ds(j * L, L)] = jnp.full((L,), 1.0, jnp.float32)
        return carry

    lax.fori_loop(0, PT_NODES // L, body2, 0)
    pltpu.sync_copy(h0_v, acc.at[ic_v], add=True)
    plsc.subcore_barrier()

    @pl.when(sid == 0)
    def _():
        pltpu.sync_copy(acc, out_hbm.at[cid])


def kernel(x, edge_index, batch, W1, b1, W2, b2):
    src = edge_index[0].astype(jnp.int32)
    dst = edge_index[1].astype(jnp.int32)
    xp = jnp.concatenate(
        [x[:, 0].astype(jnp.float32), jnp.zeros((NP - N_NODES,), jnp.float32)])
    bp = jnp.concatenate(
        [batch.astype(jnp.int32),
         jnp.full((NP - N_NODES,), PAD_GRAPH, jnp.int32)])
    W1f = W1.astype(jnp.float32)
    W2f = W2.astype(jnp.float32)
    w1t = jnp.tile(W1f.reshape(16, 1), (1, L))
    b1t = jnp.tile(b1.astype(jnp.float32).reshape(16, 1), (1, L))
    w20t = jnp.tile(W2f[:, 0].reshape(16, 1), (1, L))
    w21t = jnp.tile(W2f[:, 1].reshape(16, 1), (1, L))
    b2t = jnp.tile(b2.astype(jnp.float32).reshape(2, 1), (1, L))

    degp = _deg_kernel(dst)
    dinv2d, y2d = _dinv_y_tc(degp.reshape(2, NP // 128, 128),
                             xp.reshape(NP // 128, 128))
    dinv = dinv2d.reshape(NP)
    y = y2d.reshape(NP)
    agg1p = _edge_agg_kernel(src, dst, y)
    z0, z1 = _feat_kernel(agg1p, dinv, y, w1t, b1t, w20t, w21t)
    a0p = _edge_agg_kernel(src, dst, z0)
    a1p = _edge_agg_kernel(src, dst, z1)
    parts = _pool_kernel(a0p, a1p, z0, z1, dinv, bp, b2t)

    tot = parts[0] + parts[1]
    sums = jnp.stack([tot[0:NUM_GRAPHS], tot[64:64 + NUM_GRAPHS]], axis=1)
    cnt = tot[256:256 + NUM_GRAPHS]
    pooled = sums / jnp.clip(cnt, 1.0)[:, None]
    return jax.nn.log_softmax(pooled, axis=1)


# speedup vs baseline: 209.6638x; 1.3083x over previous
import functools

import jax
import jax.numpy as jnp
from jax import lax
from jax.experimental import pallas as pl
from jax.experimental.pallas import tpu as pltpu
from jax.experimental.pallas import tpu_sc as plsc

N_NODES = 100000
N_EDGES = 3200000
NUM_GRAPHS = 64

NC = 2
NS = 16
L = 16
NW = NC * NS

NP = 102400
PT_NODES = NP // NW
PSC_NODES = NP // NS
ET = N_EDGES // NW
C = 2000
NCHUNK = ET // C

PAD_GRAPH = 512
ACC_BINS = 1024

_MESH = plsc.VectorSubcoreMesh(core_axis_name="c", subcore_axis_name="s")
_SC_PARAMS = pltpu.CompilerParams(needs_layout_passes=False)


def _fill(ref, n, val):
    v = jnp.full((L,), val, ref.dtype)

    def body(j, carry):
        ref[pl.ds(j * L, L)] = v
        return carry

    lax.fori_loop(0, n // L, body, 0)


def _zero_shared_slice(acc, zbuf, zlen, sid, per_tile):
    base = sid * per_tile
    off = 0
    while off < per_tile:
        n = min(zlen, per_tile - off)
        pltpu.sync_copy(zbuf.at[pl.ds(0, n)], acc.at[pl.ds(base + off, n)])
        off += n


@functools.partial(
    pl.kernel,
    out_type=jax.ShapeDtypeStruct((NC, NP), jnp.float32),
    mesh=_MESH,
    compiler_params=_SC_PARAMS,
    scratch_types=[
        pltpu.VMEM((C,), jnp.int32),
        pltpu.VMEM((C,), jnp.int32),
        pltpu.VMEM((C,), jnp.int32),
        pltpu.VMEM((C,), jnp.float32),
        pltpu.SemaphoreType.DMA,
        pltpu.SemaphoreType.DMA,
        pltpu.SemaphoreType.DMA,
        pltpu.SemaphoreType.DMA,
        pltpu.SemaphoreType.DMA,
        pltpu.SemaphoreType.DMA,
        pltpu.VMEM_SHARED((NP,), jnp.float32),
    ],
)
def _deg_kernel(dst_hbm, out_hbm, dst_a, dst_b, dst_c, ones_v,
                in_a, in_b, in_c, sc_a, sc_b, sc_c, acc):
    cid = lax.axis_index("c")
    sid = lax.axis_index("s")
    wid = sid * NC + cid
    dstb = [dst_a, dst_b, dst_c]
    insem = [in_a, in_b, in_c]
    scsem = [sc_a, sc_b, sc_c]
    _fill(ones_v, C, 0.0)
    _zero_shared_slice(acc, ones_v, C, sid, PSC_NODES)
    _fill(ones_v, C, 1.0)
    plsc.subcore_barrier()
    ebase = wid * ET

    in_h = {}
    sc_h = {}
    for ci in range(min(2, NCHUNK)):
        o = pl.ds(ebase + ci * C, C)
        in_h[ci] = pltpu.async_copy(dst_hbm.at[o], dstb[ci % 3], insem[ci % 3])
    for ci in range(NCHUNK):
        cur = ci % 3
        in_h.pop(ci).wait()
        sc_h[ci] = pltpu.async_copy(ones_v, acc.at[dstb[cur]],
                                    scsem[cur], add=True)
        if ci + 2 < NCHUNK:
            if ci - 1 >= 0:
                sc_h.pop(ci - 1).wait()
            o = pl.ds(ebase + (ci + 2) * C, C)
            in_h[ci + 2] = pltpu.async_copy(dst_hbm.at[o], dstb[(ci + 2) % 3],
                                            insem[(ci + 2) % 3])
    for h in sc_h.values():
        h.wait()
    plsc.subcore_barrier()
    sl = pl.ds(sid * PSC_NODES, PSC_NODES)
    pltpu.sync_copy(acc.at[sl], out_hbm.at[cid, sl])


def _dinv_y_tc_body(degp_ref, x_ref, dinv_ref, y_ref):
    deg = degp_ref[0] + degp_ref[1] + 1.0
    dinv = lax.rsqrt(deg)
    dinv_ref[...] = dinv
    y_ref[...] = x_ref[...] * dinv


_dinv_y_tc = pl.pallas_call(
    _dinv_y_tc_body,
    out_shape=(
        jax.ShapeDtypeStruct((NP // 128, 128), jnp.float32),
        jax.ShapeDtypeStruct((NP // 128, 128), jnp.float32),
    ),
)


GU = 5


@functools.partial(
    pl.kernel,
    out_type=jax.ShapeDtypeStruct((NC, NP), jnp.float32),
    mesh=_MESH,
    compiler_params=_SC_PARAMS,
    scratch_types=[
        pltpu.VMEM((NP,), jnp.float32),
        pltpu.VMEM((C,), jnp.int32),
        pltpu.VMEM((C,), jnp.int32),
        pltpu.VMEM((C,), jnp.int32),
        pltpu.VMEM((C,), jnp.int32),
        pltpu.VMEM((C,), jnp.float32),
        pltpu.VMEM((C,), jnp.float32),
        pltpu.SemaphoreType.DMA,
        pltpu.SemaphoreType.DMA,
        pltpu.SemaphoreType.DMA,
        pltpu.SemaphoreType.DMA,
        pltpu.VMEM_SHARED((NP,), jnp.float32),
    ],
)
def _edge_agg_kernel(src_hbm, dst_hbm, tab_hbm, out_hbm,
                     tab_v, src_a, src_b, dst_a, dst_b, vals_a, vals_b,
                     in_a, in_b, sc_a, sc_b, acc):
    cid = lax.axis_index("c")
    sid = lax.axis_index("s")
    wid = sid * NC + cid
    srcb = [src_a, src_b]
    dstb = [dst_a, dst_b]
    valb = [vals_a, vals_b]
    insem = [in_a, in_b]
    scsem = [sc_a, sc_b]
    pltpu.sync_copy(tab_hbm, tab_v)
    _fill(vals_a, C, 0.0)
    _zero_shared_slice(acc, vals_a, C, sid, PSC_NODES)
    plsc.subcore_barrier()
    ebase = wid * ET

    in_h = {}
    sc_h = {}
    o = pl.ds(ebase, C)
    in_h[0] = (pltpu.async_copy(src_hbm.at[o], src_a, in_a),
               pltpu.async_copy(dst_hbm.at[o], dst_a, in_a))
    for ci in range(NCHUNK):
        cur = ci % 2
        nxt = 1 - cur
        for h in in_h.pop(ci):
            h.wait()
        if ci - 2 in sc_h:
            sc_h.pop(ci - 2).wait()

        def g(j, c2, cur=cur):
            for f in range(GU):
                s = pl.ds(j * (GU * L) + f * L, L)
                valb[cur][s] = plsc.load_gather(tab_v, [srcb[cur][s]])
            return c2

        lax.fori_loop(0, C // (GU * L), g, 0)
        if ci - 1 in sc_h:
            sc_h.pop(ci - 1).wait()
        if ci + 1 < NCHUNK:
            o = pl.ds(ebase + (ci + 1) * C, C)
            in_h[ci + 1] = (pltpu.async_copy(src_hbm.at[o], srcb[nxt],
                                             insem[nxt]),
                            pltpu.async_copy(dst_hbm.at[o], dstb[nxt],
                                             insem[nxt]))
        sc_h[ci] = pltpu.async_copy(valb[cur], acc.at[dstb[cur]],
                                    scsem[cur], add=True)
    for h in sc_h.values():
        h.wait()
    plsc.subcore_barrier()
    sl = pl.ds(sid * PSC_NODES, PSC_NODES)
    pltpu.sync_copy(acc.at[sl], out_hbm.at[cid, sl])


@functools.partial(
    pl.kernel,
    out_type=(
        jax.ShapeDtypeStruct((NP,), jnp.float32),
        jax.ShapeDtypeStruct((NP,), jnp.float32),
    ),
    mesh=_MESH,
    compiler_params=_SC_PARAMS,
    scratch_types=[
        pltpu.VMEM((PT_NODES,), jnp.float32),
        pltpu.VMEM((PT_NODES,), jnp.float32),
        pltpu.VMEM((PT_NODES,), jnp.float32),
        pltpu.VMEM((PT_NODES,), jnp.float32),
        pltpu.VMEM((PT_NODES,), jnp.float32),
        pltpu.VMEM((PT_NODES,), jnp.float32),
        pltpu.VMEM((16, L), jnp.float32),
        pltpu.VMEM((16, L), jnp.float32),
        pltpu.VMEM((16, L), jnp.float32),
        pltpu.VMEM((16, L), jnp.float32),
    ],
)
def _feat_kernel(aggp_hbm, dinv_hbm, y_hbm, w1_hbm, b1_hbm, w20_hbm, w21_hbm,
                 z0_hbm, z1_hbm,
                 p0_v, p1_v, dinv_v, y_v, z0_v, z1_v, w1_v, b1_v, w20_v, w21_v):
    cid = lax.axis_index("c")
    sid = lax.axis_index("s")
    wid = sid * NC + cid
    base = wid * PT_NODES
    sl = pl.ds(base, PT_NODES)
    pltpu.sync_copy(aggp_hbm.at[0, sl], p0_v)
    pltpu.sync_copy(aggp_hbm.at[1, sl], p1_v)
    pltpu.sync_copy(dinv_hbm.at[sl], dinv_v)
    pltpu.sync_copy(y_hbm.at[sl], y_v)
    pltpu.sync_copy(w1_hbm, w1_v)
    pltpu.sync_copy(b1_hbm, b1_v)
    pltpu.sync_copy(w20_hbm, w20_v)
    pltpu.sync_copy(w21_hbm, w21_v)

    zero = jnp.zeros((L,), jnp.float32)

    def body(j, carry):
        s = pl.ds(j * L, L)
        dv = dinv_v[s]
        s1 = dv * (p0_v[s] + p1_v[s] + y_v[s])
        g0 = zero
        g1 = zero
        for f in range(16):
            h = jnp.maximum(s1 * w1_v[f, :] + b1_v[f, :], 0.0)
            g0 = g0 + h * w20_v[f, :]
            g1 = g1 + h * w21_v[f, :]
        z0_v[s] = g0 * dv
        z1_v[s] = g1 * dv
        return carry

    lax.fori_loop(0, PT_NODES // L, body, 0)
    pltpu.sync_copy(z0_v, z0_hbm.at[sl])
    pltpu.sync_copy(z1_v, z1_hbm.at[sl])


@functools.partial(
    pl.kernel,
    out_type=jax.ShapeDtypeStruct((NC, ACC_BINS), jnp.float32),
    mesh=_MESH,
    compiler_params=_SC_PARAMS,
    scratch_types=[
        pltpu.VMEM((PT_NODES,), jnp.float32),
        pltpu.VMEM((PT_NODES,), jnp.float32),
        pltpu.VMEM((PT_NODES,), jnp.float32),
        pltpu.VMEM((PT_NODES,), jnp.float32),
        pltpu.VMEM((PT_NODES,), jnp.float32),
        pltpu.VMEM((PT_NODES,), jnp.float32),
        pltpu.VMEM((PT_NODES,), jnp.float32),
        pltpu.VMEM((PT_NODES,), jnp.int32),
        pltpu.VMEM((PT_NODES,), jnp.float32),
        pltpu.VMEM((PT_NODES,), jnp.float32),
        pltpu.VMEM((PT_NODES,), jnp.int32),
        pltpu.VMEM((PT_NODES,), jnp.int32),
        pltpu.VMEM((2, L), jnp.float32),
        pltpu.VMEM((ACC_BINS // NS,), jnp.float32),
        pltpu.VMEM_SHARED((ACC_BINS,), jnp.float32),
    ],
)
def _pool_kernel(a0p_hbm, a1p_hbm, z0_hbm, z1_hbm, dinv_hbm, batch_hbm, b2_hbm,
                 out_hbm,
                 q00_v, q01_v, q10_v, q11_v, z0_v, z1_v, dinv_v, batch_v,
                 h0_v, h1_v, ib_v, ic_v, b2_v, zb_v, acc):
    cid = lax.axis_index("c")
    sid = lax.axis_index("s")
    wid = sid * NC + cid
    base = wid * PT_NODES
    sl = pl.ds(base, PT_NODES)
    pltpu.sync_copy(a0p_hbm.at[0, sl], q00_v)
    pltpu.sync_copy(a0p_hbm.at[1, sl], q01_v)
    pltpu.sync_copy(a1p_hbm.at[0, sl], q10_v)
    pltpu.sync_copy(a1p_hbm.at[1, sl], q11_v)
    pltpu.sync_copy(z0_hbm.at[sl], z0_v)
    pltpu.sync_copy(z1_hbm.at[sl], z1_v)
    pltpu.sync_copy(dinv_hbm.at[sl], dinv_v)
    pltpu.sync_copy(batch_hbm.at[sl], batch_v)
    pltpu.sync_copy(b2_hbm, b2_v)

    _fill(zb_v, ACC_BINS // NS, 0.0)
    _zero_shared_slice(acc, zb_v, ACC_BINS // NS, sid, ACC_BINS // NS)

    k64 = jnp.full((L,), 64, jnp.int32)
    k256 = jnp.full((L,), 256, jnp.int32)

    def body(j, carry):
        s = pl.ds(j * L, L)
        dv = dinv_v[s]
        p = dv * (q00_v[s] + q01_v[s] + z0_v[s])
        q = dv * (q10_v[s] + q11_v[s] + z1_v[s])
        h0_v[s] = jnp.maximum(p + b2_v[0, :], 0.0)
        h1_v[s] = jnp.maximum(q + b2_v[1, :], 0.0)
        bi = batch_v[s]
        ib_v[s] = bi + k64
        ic_v[s] = bi + k256
        return carry

    lax.fori_loop(0, PT_NODES // L, body, 0)
    plsc.subcore_barrier()
    pltpu.sync_copy(h0_v, acc.at[batch_v], add=True)
    pltpu.sync_copy(h1_v, acc.at[ib_v], add=True)

    def body2(j, carry):
        h0_v[pl.ds(j * L, L)] = jnp.full((L,), 1.0, jnp.float32)
        return carry

    lax.fori_loop(0, PT_NODES // L, body2, 0)
    pltpu.sync_copy(h0_v, acc.at[ic_v], add=True)
    plsc.subcore_barrier()

    @pl.when(sid == 0)
    def _():
        pltpu.sync_copy(acc, out_hbm.at[cid])


def kernel(x, edge_index, batch, W1, b1, W2, b2):
    src = edge_index[0].astype(jnp.int32)
    dst = edge_index[1].astype(jnp.int32)
    xp = jnp.concatenate(
        [x[:, 0].astype(jnp.float32), jnp.zeros((NP - N_NODES,), jnp.float32)])
    bp = jnp.concatenate(
        [batch.astype(jnp.int32),
         jnp.full((NP - N_NODES,), PAD_GRAPH, jnp.int32)])
    W1f = W1.astype(jnp.float32)
    W2f = W2.astype(jnp.float32)
    w1t = jnp.tile(W1f.reshape(16, 1), (1, L))
    b1t = jnp.tile(b1.astype(jnp.float32).reshape(16, 1), (1, L))
    w20t = jnp.tile(W2f[:, 0].reshape(16, 1), (1, L))
    w21t = jnp.tile(W2f[:, 1].reshape(16, 1), (1, L))
    b2t = jnp.tile(b2.astype(jnp.float32).reshape(2, 1), (1, L))

    degp = _deg_kernel(dst)
    dinv2d, y2d = _dinv_y_tc(degp.reshape(2, NP // 128, 128),
                             xp.reshape(NP // 128, 128))
    dinv = dinv2d.reshape(NP)
    y = y2d.reshape(NP)
    agg1p = _edge_agg_kernel(src, dst, y)
    z0, z1 = _feat_kernel(agg1p, dinv, y, w1t, b1t, w20t, w21t)
    a0p = _edge_agg_kernel(src, dst, z0)
    a1p = _edge_agg_kernel(src, dst, z1)
    parts = _pool_kernel(a0p, a1p, z0, z1, dinv, bp, b2t)

    tot = parts[0] + parts[1]
    sums = jnp.stack([tot[0:NUM_GRAPHS], tot[64:64 + NUM_GRAPHS]], axis=1)
    cnt = tot[256:256 + NUM_GRAPHS]
    pooled = sums / jnp.clip(cnt, 1.0)[:, None]
    return jax.nn.log_softmax(pooled, axis=1)


# speedup vs baseline: 252.2778x; 1.2032x over previous
import functools

import jax
import jax.numpy as jnp
from jax import lax
from jax.experimental import pallas as pl
from jax.experimental.pallas import tpu as pltpu
from jax.experimental.pallas import tpu_sc as plsc

N_NODES = 100000
N_EDGES = 3200000
NUM_GRAPHS = 64

NC = 2
NS = 16
L = 16
NW = NC * NS

NP = 102400
PT_NODES = NP // NW
PSC_NODES = NP // NS
ET = N_EDGES // NW

PAD_GRAPH = 512
ACC_BINS = 1024

_MESH = plsc.VectorSubcoreMesh(core_axis_name="c", subcore_axis_name="s")
_SC_PARAMS = pltpu.CompilerParams(needs_layout_passes=False)

MASK_HI = -65536


def _fill(ref, n, val):
    v = jnp.full((L,), val, ref.dtype)

    def body(j, carry):
        ref[pl.ds(j * L, L)] = v
        return carry

    lax.fori_loop(0, n // L, body, 0)


def _zero_shared_slice(acc, zbuf, zlen, sid, per_tile):
    base = sid * per_tile
    off = 0
    while off < per_tile:
        n = min(zlen, per_tile - off)
        pltpu.sync_copy(zbuf.at[pl.ds(0, n)], acc.at[pl.ds(base + off, n)])
        off += n


CD = 2000
NCD = ET // CD


@functools.partial(
    pl.kernel,
    out_type=jax.ShapeDtypeStruct((NC, NP), jnp.float32),
    mesh=_MESH,
    compiler_params=_SC_PARAMS,
    scratch_types=[
        pltpu.VMEM((CD,), jnp.int32),
        pltpu.VMEM((CD,), jnp.int32),
        pltpu.VMEM((CD,), jnp.int32),
        pltpu.VMEM((CD,), jnp.float32),
        pltpu.SemaphoreType.DMA,
        pltpu.SemaphoreType.DMA,
        pltpu.SemaphoreType.DMA,
        pltpu.SemaphoreType.DMA,
        pltpu.SemaphoreType.DMA,
        pltpu.SemaphoreType.DMA,
        pltpu.VMEM_SHARED((NP,), jnp.float32),
    ],
)
def _deg_kernel(dst_hbm, out_hbm, dst_a, dst_b, dst_c, ones_v,
                in_a, in_b, in_c, sc_a, sc_b, sc_c, acc):
    cid = lax.axis_index("c")
    sid = lax.axis_index("s")
    wid = sid * NC + cid
    dstb = [dst_a, dst_b, dst_c]
    insem = [in_a, in_b, in_c]
    scsem = [sc_a, sc_b, sc_c]
    _fill(ones_v, CD, 0.0)
    _zero_shared_slice(acc, ones_v, CD, sid, PSC_NODES)
    _fill(ones_v, CD, 1.0)
    plsc.subcore_barrier()
    ebase = wid * ET

    in_h = {}
    sc_h = {}
    for ci in range(min(2, NCD)):
        o = pl.ds(ebase + ci * CD, CD)
        in_h[ci] = pltpu.async_copy(dst_hbm.at[o], dstb[ci % 3], insem[ci % 3])
    for ci in range(NCD):
        cur = ci % 3
        in_h.pop(ci).wait()
        sc_h[ci] = pltpu.async_copy(ones_v, acc.at[dstb[cur]],
                                    scsem[cur], add=True)
        if ci + 2 < NCD:
            if ci - 1 >= 0:
                sc_h.pop(ci - 1).wait()
            o = pl.ds(ebase + (ci + 2) * CD, CD)
            in_h[ci + 2] = pltpu.async_copy(dst_hbm.at[o], dstb[(ci + 2) % 3],
                                            insem[(ci + 2) % 3])
    for h in sc_h.values():
        h.wait()
    plsc.subcore_barrier()
    sl = pl.ds(sid * PSC_NODES, PSC_NODES)
    pltpu.sync_copy(acc.at[sl], out_hbm.at[cid, sl])


def _dinv_y_tc_body(degp_ref, x_ref, dinv_ref, y_ref):
    deg = degp_ref[0] + degp_ref[1] + 1.0
    dinv = lax.rsqrt(deg)
    dinv_ref[...] = dinv
    y_ref[...] = x_ref[...] * dinv


_dinv_y_tc = pl.pallas_call(
    _dinv_y_tc_body,
    out_shape=(
        jax.ShapeDtypeStruct((NP // 128, 128), jnp.float32),
        jax.ShapeDtypeStruct((NP // 128, 128), jnp.float32),
    ),
)


CA = 4000
NCA = ET // CA
GUA = 10


@functools.partial(
    pl.kernel,
    out_type=jax.ShapeDtypeStruct((NC, NP), jnp.float32),
    mesh=_MESH,
    compiler_params=_SC_PARAMS,
    scratch_types=[
        pltpu.VMEM((NP,), jnp.float32),
        pltpu.VMEM((CA,), jnp.int32),
        pltpu.VMEM((CA,), jnp.int32),
        pltpu.VMEM((CA,), jnp.int32),
        pltpu.VMEM((CA,), jnp.float32),
        pltpu.VMEM((CA,), jnp.float32),
        pltpu.SemaphoreType.DMA,
        pltpu.SemaphoreType.DMA,
        pltpu.SemaphoreType.DMA,
        pltpu.SemaphoreType.DMA,
        pltpu.SemaphoreType.DMA,
        pltpu.VMEM_SHARED((NP,), jnp.float32),
    ],
)
def _edge_agg_kernel(src_hbm, dst_hbm, tab_hbm, out_hbm,
                     tab_v, src_v, dst_a, dst_b, vals_a, vals_b,
                     in_s, in_a, in_b, sc_a, sc_b, acc):
    cid = lax.axis_index("c")
    sid = lax.axis_index("s")
    wid = sid * NC + cid
    dstb = [dst_a, dst_b]
    valb = [vals_a, vals_b]
    insem = [in_a, in_b]
    scsem = [sc_a, sc_b]
    pltpu.sync_copy(tab_hbm, tab_v)
    _fill(vals_a, CA, 0.0)
    _zero_shared_slice(acc, vals_a, CA, sid, PSC_NODES)
    plsc.subcore_barrier()
    ebase = wid * ET

    sc_h = {}
    o = pl.ds(ebase, CA)
    src_h = pltpu.async_copy(src_hbm.at[o], src_v, in_s)
    dst_h = {0: pltpu.async_copy(dst_hbm.at[o], dst_a, in_a)}
    for ci in range(NCA):
        cur = ci % 2
        nxt = 1 - cur
        src_h.wait()
        dst_h.pop(ci).wait()
        if ci - 2 in sc_h:
            sc_h.pop(ci - 2).wait()

        def g(j, c2, cur=cur):
            for f in range(GUA):
                s = pl.ds(j * (GUA * L) + f * L, L)
                valb[cur][s] = plsc.load_gather(tab_v, [src_v[s]])
            return c2

        lax.fori_loop(0, CA // (GUA * L), g, 0)
        if ci + 1 < NCA:
            o = pl.ds(ebase + (ci + 1) * CA, CA)
            src_h = pltpu.async_copy(src_hbm.at[o], src_v, in_s)
            if ci - 1 in sc_h:
                sc_h.pop(ci - 1).wait()
            dst_h[ci + 1] = pltpu.async_copy(dst_hbm.at[o], dstb[nxt],
                                             insem[nxt])
        sc_h[ci] = pltpu.async_copy(valb[cur], acc.at[dstb[cur]],
                                    scsem[cur], add=True)
    for h in sc_h.values():
        h.wait()
    plsc.subcore_barrier()
    sl = pl.ds(sid * PSC_NODES, PSC_NODES)
    pltpu.sync_copy(acc.at[sl], out_hbm.at[cid, sl])


@functools.partial(
    pl.kernel,
    out_type=jax.ShapeDtypeStruct((NP,), jnp.float32),
    mesh=_MESH,
    compiler_params=_SC_PARAMS,
    scratch_types=[
        pltpu.VMEM((PT_NODES,), jnp.float32),
        pltpu.VMEM((PT_NODES,), jnp.float32),
        pltpu.VMEM((PT_NODES,), jnp.float32),
        pltpu.VMEM((PT_NODES,), jnp.float32),
        pltpu.VMEM((PT_NODES,), jnp.float32),
        pltpu.VMEM((16, L), jnp.float32),
        pltpu.VMEM((16, L), jnp.float32),
        pltpu.VMEM((16, L), jnp.float32),
        pltpu.VMEM((16, L), jnp.float32),
    ],
)
def _feat_kernel(aggp_hbm, dinv_hbm, y_hbm, w1_hbm, b1_hbm, w20_hbm, w21_hbm,
                 z01_hbm,
                 p0_v, p1_v, dinv_v, y_v, z01_v, w1_v, b1_v, w20_v, w21_v):
    cid = lax.axis_index("c")
    sid = lax.axis_index("s")
    wid = sid * NC + cid
    base = wid * PT_NODES
    sl = pl.ds(base, PT_NODES)
    pltpu.sync_copy(aggp_hbm.at[0, sl], p0_v)
    pltpu.sync_copy(aggp_hbm.at[1, sl], p1_v)
    pltpu.sync_copy(dinv_hbm.at[sl], dinv_v)
    pltpu.sync_copy(y_hbm.at[sl], y_v)
    pltpu.sync_copy(w1_hbm, w1_v)
    pltpu.sync_copy(b1_hbm, b1_v)
    pltpu.sync_copy(w20_hbm, w20_v)
    pltpu.sync_copy(w21_hbm, w21_v)

    zero = jnp.zeros((L,), jnp.float32)
    mhi = jnp.full((L,), MASK_HI, jnp.int32)

    def body(j, carry):
        s = pl.ds(j * L, L)
        dv = dinv_v[s]
        s1 = dv * (p0_v[s] + p1_v[s] + y_v[s])
        g0 = zero
        g1 = zero
        for f in range(16):
            h = jnp.maximum(s1 * w1_v[f, :] + b1_v[f, :], 0.0)
            g0 = g0 + h * w20_v[f, :]
            g1 = g1 + h * w21_v[f, :]
        i0 = plsc.bitcast(g0 * dv, jnp.int32) & mhi
        i1 = lax.shift_right_logical(plsc.bitcast(g1 * dv, jnp.int32), 16)
        z01_v[s] = plsc.bitcast(i0 | i1, jnp.float32)
        return carry

    lax.fori_loop(0, PT_NODES // L, body, 0)
    pltpu.sync_copy(z01_v, z01_hbm.at[sl])


CB = 2000
NCB = ET // CB
GUB = 5


@functools.partial(
    pl.kernel,
    out_type=jax.ShapeDtypeStruct((NC, 2, NP), jnp.float32),
    mesh=_MESH,
    compiler_params=_SC_PARAMS,
    scratch_types=[
        pltpu.VMEM((NP,), jnp.float32),
        pltpu.VMEM((CB,), jnp.int32),
        pltpu.VMEM((CB,), jnp.int32),
        pltpu.VMEM((CB,), jnp.int32),
        pltpu.VMEM((CB,), jnp.float32),
        pltpu.VMEM((CB,), jnp.float32),
        pltpu.VMEM((CB,), jnp.float32),
        pltpu.VMEM((CB,), jnp.float32),
        pltpu.SemaphoreType.DMA,
        pltpu.SemaphoreType.DMA,
        pltpu.SemaphoreType.DMA,
        pltpu.SemaphoreType.DMA,
        pltpu.SemaphoreType.DMA,
        pltpu.SemaphoreType.DMA,
        pltpu.SemaphoreType.DMA,
        pltpu.VMEM_SHARED((NP,), jnp.float32),
        pltpu.VMEM_SHARED((NP,), jnp.float32),
    ],
)
def _edge_agg2_kernel(src_hbm, dst_hbm, tab_hbm, out_hbm,
                      tab_v, src_v, dst_a, dst_b, v0_a, v0_b, v1_a, v1_b,
                      in_s, in_a, in_b, s0_a, s0_b, s1_a, s1_b,
                      acc0, acc1):
    cid = lax.axis_index("c")
    sid = lax.axis_index("s")
    wid = sid * NC + cid
    dstb = [dst_a, dst_b]
    v0b = [v0_a, v0_b]
    v1b = [v1_a, v1_b]
    insem = [in_a, in_b]
    s0sem = [s0_a, s0_b]
    s1sem = [s1_a, s1_b]
    pltpu.sync_copy(tab_hbm, tab_v)
    _fill(v0_a, CB, 0.0)
    _zero_shared_slice(acc0, v0_a, CB, sid, PSC_NODES)
    _zero_shared_slice(acc1, v0_a, CB, sid, PSC_NODES)
    plsc.subcore_barrier()
    ebase = wid * ET
    mhi = jnp.full((L,), MASK_HI, jnp.int32)

    sc_h = {}
    o = pl.ds(ebase, CB)
    src_h = pltpu.async_copy(src_hbm.at[o], src_v, in_s)
    dst_h = {0: pltpu.async_copy(dst_hbm.at[o], dst_a, in_a)}
    for ci in range(NCB):
        cur = ci % 2
        nxt = 1 - cur
        src_h.wait()
        dst_h.pop(ci).wait()
        if ci - 2 in sc_h:
            for h in sc_h.pop(ci - 2):
                h.wait()

        def g(j, c2, cur=cur):
            for f in range(GUB):
                s = pl.ds(j * (GUB * L) + f * L, L)
                i = plsc.bitcast(plsc.load_gather(tab_v, [src_v[s]]),
                                 jnp.int32)
                v0b[cur][s] = plsc.bitcast(i & mhi, jnp.float32)
                v1b[cur][s] = plsc.bitcast(lax.shift_left(i, 16), jnp.float32)
            return c2

        lax.fori_loop(0, CB // (GUB * L), g, 0)
        if ci + 1 < NCB:
            o = pl.ds(ebase + (ci + 1) * CB, CB)
            src_h = pltpu.async_copy(src_hbm.at[o], src_v, in_s)
            if ci - 1 in sc_h:
                for h in sc_h.pop(ci - 1):
                    h.wait()
            dst_h[ci + 1] = pltpu.async_copy(dst_hbm.at[o], dstb[nxt],
                                             insem[nxt])
        sc_h[ci] = (
            pltpu.async_copy(v0b[cur], acc0.at[dstb[cur]], s0sem[cur],
                             add=True),
            pltpu.async_copy(v1b[cur], acc1.at[dstb[cur]], s1sem[cur],
                             add=True),
        )
    for hs in sc_h.values():
        for h in hs:
            h.wait()
    plsc.subcore_barrier()
    sl = pl.ds(sid * PSC_NODES, PSC_NODES)
    pltpu.sync_copy(acc0.at[sl], out_hbm.at[cid, 0, sl])
    pltpu.sync_copy(acc1.at[sl], out_hbm.at[cid, 1, sl])


@functools.partial(
    pl.kernel,
    out_type=jax.ShapeDtypeStruct((NC, ACC_BINS), jnp.float32),
    mesh=_MESH,
    compiler_params=_SC_PARAMS,
    scratch_types=[
        pltpu.VMEM((PT_NODES,), jnp.float32),
        pltpu.VMEM((PT_NODES,), jnp.float32),
        pltpu.VMEM((PT_NODES,), jnp.float32),
        pltpu.VMEM((PT_NODES,), jnp.float32),
        pltpu.VMEM((PT_NODES,), jnp.float32),
        pltpu.VMEM((PT_NODES,), jnp.float32),
        pltpu.VMEM((PT_NODES,), jnp.int32),
        pltpu.VMEM((PT_NODES,), jnp.float32),
        pltpu.VMEM((PT_NODES,), jnp.float32),
        pltpu.VMEM((PT_NODES,), jnp.int32),
        pltpu.VMEM((PT_NODES,), jnp.int32),
        pltpu.VMEM((2, L), jnp.float32),
        pltpu.VMEM((ACC_BINS // NS,), jnp.float32),
        pltpu.VMEM_SHARED((ACC_BINS,), jnp.float32),
    ],
)
def _pool_kernel(ap_hbm, z01_hbm, dinv_hbm, batch_hbm, b2_hbm,
                 out_hbm,
                 q00_v, q01_v, q10_v, q11_v, z01_v, dinv_v, batch_v,
                 h0_v, h1_v, ib_v, ic_v, b2_v, zb_v, acc):
    cid = lax.axis_index("c")
    sid = lax.axis_index("s")
    wid = sid * NC + cid
    base = wid * PT_NODES
    sl = pl.ds(base, PT_NODES)
    pltpu.sync_copy(ap_hbm.at[0, 0, sl], q00_v)
    pltpu.sync_copy(ap_hbm.at[1, 0, sl], q01_v)
    pltpu.sync_copy(ap_hbm.at[0, 1, sl], q10_v)
    pltpu.sync_copy(ap_hbm.at[1, 1, sl], q11_v)
    pltpu.sync_copy(z01_hbm.at[sl], z01_v)
    pltpu.sync_copy(dinv_hbm.at[sl], dinv_v)
    pltpu.sync_copy(batch_hbm.at[sl], batch_v)
    pltpu.sync_copy(b2_hbm, b2_v)

    _fill(zb_v, ACC_BINS // NS, 0.0)
    _zero_shared_slice(acc, zb_v, ACC_BINS // NS, sid, ACC_BINS // NS)

    k64 = jnp.full((L,), 64, jnp.int32)
    k256 = jnp.full((L,), 256, jnp.int32)
    mhi = jnp.full((L,), MASK_HI, jnp.int32)

    def body(j, carry):
        s = pl.ds(j * L, L)
        dv = dinv_v[s]
        i = plsc.bitcast(z01_v[s], jnp.int32)
        z0 = plsc.bitcast(i & mhi, jnp.float32)
        z1 = plsc.bitcast(lax.shift_left(i, 16), jnp.float32)
        p = dv * (q00_v[s] + q01_v[s] + z0)
        q = dv * (q10_v[s] + q11_v[s] + z1)
        h0_v[s] = jnp.maximum(p + b2_v[0, :], 0.0)
        h1_v[s] = jnp.maximum(q + b2_v[1, :], 0.0)
        bi = batch_v[s]
        ib_v[s] = bi + k64
        ic_v[s] = bi + k256
        return carry

    lax.fori_loop(0, PT_NODES // L, body, 0)
    plsc.subcore_barrier()
    pltpu.sync_copy(h0_v, acc.at[batch_v], add=True)
    pltpu.sync_copy(h1_v, acc.at[ib_v], add=True)

    def body2(j, carry):
        h0_v[pl.ds(j * L, L)] = jnp.full((L,), 1.0, jnp.float32)
        return carry

    lax.fori_loop(0, PT_NODES // L, body2, 0)
    pltpu.sync_copy(h0_v, acc.at[ic_v], add=True)
    plsc.subcore_barrier()

    @pl.when(sid == 0)
    def _():
        pltpu.sync_copy(acc, out_hbm.at[cid])


def kernel(x, edge_index, batch, W1, b1, W2, b2):
    src = edge_index[0].astype(jnp.int32)
    dst = edge_index[1].astype(jnp.int32)
    xp = jnp.concatenate(
        [x[:, 0].astype(jnp.float32), jnp.zeros((NP - N_NODES,), jnp.float32)])
    bp = jnp.concatenate(
        [batch.astype(jnp.int32),
         jnp.full((NP - N_NODES,), PAD_GRAPH, jnp.int32)])
    W1f = W1.astype(jnp.float32)
    W2f = W2.astype(jnp.float32)
    w1t = jnp.tile(W1f.reshape(16, 1), (1, L))
    b1t = jnp.tile(b1.astype(jnp.float32).reshape(16, 1), (1, L))
    w20t = jnp.tile(W2f[:, 0].reshape(16, 1), (1, L))
    w21t = jnp.tile(W2f[:, 1].reshape(16, 1), (1, L))
    b2t = jnp.tile(b2.astype(jnp.float32).reshape(2, 1), (1, L))

    degp = _deg_kernel(dst)
    dinv2d, y2d = _dinv_y_tc(degp.reshape(2, NP // 128, 128),
                             xp.reshape(NP // 128, 128))
    dinv = dinv2d.reshape(NP)
    y = y2d.reshape(NP)
    agg1p = _edge_agg_kernel(src, dst, y)
    z01 = _feat_kernel(agg1p, dinv, y, w1t, b1t, w20t, w21t)
    ap = _edge_agg2_kernel(src, dst, z01)
    parts = _pool_kernel(ap, z01, dinv, bp, b2t)

    tot = parts[0] + parts[1]
    sums = jnp.stack([tot[0:NUM_GRAPHS], tot[64:64 + NUM_GRAPHS]], axis=1)
    cnt = tot[256:256 + NUM_GRAPHS]
    pooled = sums / jnp.clip(cnt, 1.0)[:, None]
    return jax.nn.log_softmax(pooled, axis=1)


# speedup vs baseline: 257.4322x; 1.0204x over previous
import functools

import jax
import jax.numpy as jnp
from jax import lax
from jax.experimental import pallas as pl
from jax.experimental.pallas import tpu as pltpu
from jax.experimental.pallas import tpu_sc as plsc

N_NODES = 100000
N_EDGES = 3200000
NUM_GRAPHS = 64

NC = 2
NS = 16
L = 16
NW = NC * NS

NP = 102400
PT_NODES = NP // NW
PSC_NODES = NP // NS
ET = N_EDGES // NW

TN = N_NODES
PAD_GRAPH = 512
ACC_BINS = 1024

_MESH = plsc.VectorSubcoreMesh(core_axis_name="c", subcore_axis_name="s")
_SC_PARAMS = pltpu.CompilerParams(needs_layout_passes=False)

MASK_HI = -65536


def _fill(ref, n, val):
    v = jnp.full((L,), val, ref.dtype)

    def body(j, carry):
        ref[pl.ds(j * L, L)] = v
        return carry

    lax.fori_loop(0, n // L, body, 0)


def _zero_shared_slice(acc, zbuf, zlen, sid, per_tile):
    base = sid * per_tile
    off = 0
    while off < per_tile:
        n = min(zlen, per_tile - off)
        pltpu.sync_copy(zbuf.at[pl.ds(0, n)], acc.at[pl.ds(base + off, n)])
        off += n


CD = 2000
NCD = ET // CD


@functools.partial(
    pl.kernel,
    out_type=jax.ShapeDtypeStruct((NC, NP), jnp.float32),
    mesh=_MESH,
    compiler_params=_SC_PARAMS,
    scratch_types=[
        pltpu.VMEM((CD,), jnp.int32),
        pltpu.VMEM((CD,), jnp.int32),
        pltpu.VMEM((CD,), jnp.int32),
        pltpu.VMEM((CD,), jnp.float32),
        pltpu.SemaphoreType.DMA,
        pltpu.SemaphoreType.DMA,
        pltpu.SemaphoreType.DMA,
        pltpu.SemaphoreType.DMA,
        pltpu.SemaphoreType.DMA,
        pltpu.SemaphoreType.DMA,
        pltpu.VMEM_SHARED((NP,), jnp.float32),
    ],
)
def _deg_kernel(dst_hbm, out_hbm, dst_a, dst_b, dst_c, ones_v,
                in_a, in_b, in_c, sc_a, sc_b, sc_c, acc):
    cid = lax.axis_index("c")
    sid = lax.axis_index("s")
    wid = sid * NC + cid
    dstb = [dst_a, dst_b, dst_c]
    insem = [in_a, in_b, in_c]
    scsem = [sc_a, sc_b, sc_c]
    _fill(ones_v, CD, 0.0)
    _zero_shared_slice(acc, ones_v, CD, sid, PSC_NODES)
    _fill(ones_v, CD, 1.0)
    plsc.subcore_barrier()
    ebase = wid * ET

    in_h = {}
    sc_h = {}
    for ci in range(min(2, NCD)):
        o = pl.ds(ebase + ci * CD, CD)
        in_h[ci] = pltpu.async_copy(dst_hbm.at[o], dstb[ci % 3], insem[ci % 3])
    for ci in range(NCD):
        cur = ci % 3
        in_h.pop(ci).wait()
        sc_h[ci] = pltpu.async_copy(ones_v, acc.at[dstb[cur]],
                                    scsem[cur], add=True)
        if ci + 2 < NCD:
            if ci - 1 >= 0:
                sc_h.pop(ci - 1).wait()
            o = pl.ds(ebase + (ci + 2) * CD, CD)
            in_h[ci + 2] = pltpu.async_copy(dst_hbm.at[o], dstb[(ci + 2) % 3],
                                            insem[(ci + 2) % 3])
    for h in sc_h.values():
        h.wait()
    plsc.subcore_barrier()
    sl = pl.ds(sid * PSC_NODES, PSC_NODES)
    pltpu.sync_copy(acc.at[sl], out_hbm.at[cid, sl])


def _dinv_y_tc_body(degp_ref, x_ref, dinv_ref, y_ref):
    deg = degp_ref[0] + degp_ref[1] + 1.0
    dinv = lax.rsqrt(deg)
    dinv_ref[...] = dinv
    y_ref[...] = x_ref[...] * dinv


_dinv_y_tc = pl.pallas_call(
    _dinv_y_tc_body,
    out_shape=(
        jax.ShapeDtypeStruct((NP // 128, 128), jnp.float32),
        jax.ShapeDtypeStruct((NP // 128, 128), jnp.float32),
    ),
)


CA = 4000
NCA = ET // CA
GUA = 10


@functools.partial(
    pl.kernel,
    out_type=jax.ShapeDtypeStruct((NC, NP), jnp.float32),
    mesh=_MESH,
    compiler_params=_SC_PARAMS,
    scratch_types=[
        pltpu.VMEM((TN,), jnp.float32),
        pltpu.VMEM((CA,), jnp.int32),
        pltpu.VMEM((CA,), jnp.int32),
        pltpu.VMEM((CA,), jnp.int32),
        pltpu.VMEM((CA,), jnp.int32),
        pltpu.VMEM((CA,), jnp.float32),
        pltpu.VMEM((CA,), jnp.float32),
        pltpu.SemaphoreType.DMA,
        pltpu.SemaphoreType.DMA,
        pltpu.SemaphoreType.DMA,
        pltpu.SemaphoreType.DMA,
        pltpu.SemaphoreType.DMA,
        pltpu.SemaphoreType.DMA,
        pltpu.VMEM_SHARED((NP,), jnp.float32),
    ],
)
def _edge_agg_kernel(src_hbm, dst_hbm, tab_hbm, out_hbm,
                     tab_v, src_a, src_b, dst_a, dst_b, vals_a, vals_b,
                     in_sa, in_sb, in_a, in_b, sc_a, sc_b, acc):
    cid = lax.axis_index("c")
    sid = lax.axis_index("s")
    wid = sid * NC + cid
    srcb = [src_a, src_b]
    dstb = [dst_a, dst_b]
    valb = [vals_a, vals_b]
    srcsem = [in_sa, in_sb]
    insem = [in_a, in_b]
    scsem = [sc_a, sc_b]
    pltpu.sync_copy(tab_hbm.at[pl.ds(0, TN)], tab_v)
    _fill(vals_a, CA, 0.0)
    _zero_shared_slice(acc, vals_a, CA, sid, PSC_NODES)
    plsc.subcore_barrier()
    ebase = wid * ET

    sc_h = {}
    o = pl.ds(ebase, CA)
    in_h = {0: (pltpu.async_copy(src_hbm.at[o], src_a, in_sa),
                pltpu.async_copy(dst_hbm.at[o], dst_a, in_a))}
    for ci in range(NCA):
        cur = ci % 2
        nxt = 1 - cur
        for h in in_h.pop(ci):
            h.wait()
        if ci - 2 in sc_h:
            sc_h.pop(ci - 2).wait()

        def g(j, c2, cur=cur):
            for f in range(GUA):
                s = pl.ds(j * (GUA * L) + f * L, L)
                valb[cur][s] = plsc.load_gather(tab_v, [srcb[cur][s]])
            return c2

        lax.fori_loop(0, CA // (GUA * L), g, 0)
        if ci + 1 < NCA:
            o = pl.ds(ebase + (ci + 1) * CA, CA)
            if ci - 1 in sc_h:
                sc_h.pop(ci - 1).wait()
            in_h[ci + 1] = (pltpu.async_copy(src_hbm.at[o], srcb[nxt],
                                             srcsem[nxt]),
                            pltpu.async_copy(dst_hbm.at[o], dstb[nxt],
                                             insem[nxt]))
        sc_h[ci] = pltpu.async_copy(valb[cur], acc.at[dstb[cur]],
                                    scsem[cur], add=True)
    for h in sc_h.values():
        h.wait()
    plsc.subcore_barrier()
    sl = pl.ds(sid * PSC_NODES, PSC_NODES)
    pltpu.sync_copy(acc.at[sl], out_hbm.at[cid, sl])


@functools.partial(
    pl.kernel,
    out_type=jax.ShapeDtypeStruct((NP,), jnp.float32),
    mesh=_MESH,
    compiler_params=_SC_PARAMS,
    scratch_types=[
        pltpu.VMEM((PT_NODES,), jnp.float32),
        pltpu.VMEM((PT_NODES,), jnp.float32),
        pltpu.VMEM((PT_NODES,), jnp.float32),
        pltpu.VMEM((PT_NODES,), jnp.float32),
        pltpu.VMEM((PT_NODES,), jnp.float32),
        pltpu.VMEM((16, L), jnp.float32),
        pltpu.VMEM((16, L), jnp.float32),
        pltpu.VMEM((16, L), jnp.float32),
        pltpu.VMEM((16, L), jnp.float32),
        pltpu.SemaphoreType.DMA,
    ],
)
def _feat_kernel(aggp_hbm, dinv_hbm, y_hbm, w1_hbm, b1_hbm, w20_hbm, w21_hbm,
                 z01_hbm,
                 p0_v, p1_v, dinv_v, y_v, z01_v, w1_v, b1_v, w20_v, w21_v,
                 sem):
    cid = lax.axis_index("c")
    sid = lax.axis_index("s")
    wid = sid * NC + cid
    base = wid * PT_NODES
    sl = pl.ds(base, PT_NODES)
    hs = [pltpu.async_copy(aggp_hbm.at[0, sl], p0_v, sem),
          pltpu.async_copy(aggp_hbm.at[1, sl], p1_v, sem),
          pltpu.async_copy(dinv_hbm.at[sl], dinv_v, sem),
          pltpu.async_copy(y_hbm.at[sl], y_v, sem),
          pltpu.async_copy(w1_hbm, w1_v, sem),
          pltpu.async_copy(b1_hbm, b1_v, sem),
          pltpu.async_copy(w20_hbm, w20_v, sem),
          pltpu.async_copy(w21_hbm, w21_v, sem)]
    for h in hs:
        h.wait()

    zero = jnp.zeros((L,), jnp.float32)
    mhi = jnp.full((L,), MASK_HI, jnp.int32)

    def body(j, carry):
        s = pl.ds(j * L, L)
        dv = dinv_v[s]
        s1 = dv * (p0_v[s] + p1_v[s] + y_v[s])
        g0 = zero
        g1 = zero
        for f in range(16):
            h = jnp.maximum(s1 * w1_v[f, :] + b1_v[f, :], 0.0)
            g0 = g0 + h * w20_v[f, :]
            g1 = g1 + h * w21_v[f, :]
        i0 = plsc.bitcast(g0 * dv, jnp.int32) & mhi
        i1 = lax.shift_right_logical(plsc.bitcast(g1 * dv, jnp.int32), 16)
        z01_v[s] = plsc.bitcast(i0 | i1, jnp.float32)
        return carry

    lax.fori_loop(0, PT_NODES // L, body, 0)
    pltpu.sync_copy(z01_v, z01_hbm.at[sl])


CB = 2000
NCB = ET // CB
GUB = 5


@functools.partial(
    pl.kernel,
    out_type=jax.ShapeDtypeStruct((NC, 2, NP), jnp.float32),
    mesh=_MESH,
    compiler_params=_SC_PARAMS,
    scratch_types=[
        pltpu.VMEM((TN,), jnp.float32),
        pltpu.VMEM((CB,), jnp.int32),
        pltpu.VMEM((CB,), jnp.int32),
        pltpu.VMEM((CB,), jnp.int32),
        pltpu.VMEM((CB,), jnp.int32),
        pltpu.VMEM((CB,), jnp.float32),
        pltpu.VMEM((CB,), jnp.float32),
        pltpu.VMEM((CB,), jnp.float32),
        pltpu.VMEM((CB,), jnp.float32),
        pltpu.SemaphoreType.DMA,
        pltpu.SemaphoreType.DMA,
        pltpu.SemaphoreType.DMA,
        pltpu.SemaphoreType.DMA,
        pltpu.SemaphoreType.DMA,
        pltpu.SemaphoreType.DMA,
        pltpu.SemaphoreType.DMA,
        pltpu.SemaphoreType.DMA,
        pltpu.VMEM_SHARED((NP,), jnp.float32),
        pltpu.VMEM_SHARED((NP,), jnp.float32),
    ],
)
def _edge_agg2_kernel(src_hbm, dst_hbm, tab_hbm, out_hbm,
                      tab_v, src_a, src_b, dst_a, dst_b, v0_a, v0_b, v1_a, v1_b,
                      in_sa, in_sb, in_a, in_b, s0_a, s0_b, s1_a, s1_b,
                      acc0, acc1):
    cid = lax.axis_index("c")
    sid = lax.axis_index("s")
    wid = sid * NC + cid
    srcb = [src_a, src_b]
    dstb = [dst_a, dst_b]
    v0b = [v0_a, v0_b]
    v1b = [v1_a, v1_b]
    srcsem = [in_sa, in_sb]
    insem = [in_a, in_b]
    s0sem = [s0_a, s0_b]
    s1sem = [s1_a, s1_b]
    pltpu.sync_copy(tab_hbm.at[pl.ds(0, TN)], tab_v)
    _fill(v0_a, CB, 0.0)
    _zero_shared_slice(acc0, v0_a, CB, sid, PSC_NODES)
    _zero_shared_slice(acc1, v0_a, CB, sid, PSC_NODES)
    plsc.subcore_barrier()
    ebase = wid * ET
    mhi = jnp.full((L,), MASK_HI, jnp.int32)

    sc_h = {}
    o = pl.ds(ebase, CB)
    in_h = {0: (pltpu.async_copy(src_hbm.at[o], src_a, in_sa),
                pltpu.async_copy(dst_hbm.at[o], dst_a, in_a))}
    for ci in range(NCB):
        cur = ci % 2
        nxt = 1 - cur
        for h in in_h.pop(ci):
            h.wait()
        if ci - 2 in sc_h:
            for h in sc_h.pop(ci - 2):
                h.wait()

        def g(j, c2, cur=cur):
            for f in range(GUB):
                s = pl.ds(j * (GUB * L) + f * L, L)
                i = plsc.bitcast(plsc.load_gather(tab_v, [srcb[cur][s]]),
                                 jnp.int32)
                v0b[cur][s] = plsc.bitcast(i & mhi, jnp.float32)
                v1b[cur][s] = plsc.bitcast(lax.shift_left(i, 16), jnp.float32)
            return c2

        lax.fori_loop(0, CB // (GUB * L), g, 0)
        if ci + 1 < NCB:
            o = pl.ds(ebase + (ci + 1) * CB, CB)
            if ci - 1 in sc_h:
                for h in sc_h.pop(ci - 1):
                    h.wait()
            in_h[ci + 1] = (pltpu.async_copy(src_hbm.at[o], srcb[nxt],
                                             srcsem[nxt]),
                            pltpu.async_copy(dst_hbm.at[o], dstb[nxt],
                                             insem[nxt]))
        sc_h[ci] = (
            pltpu.async_copy(v0b[cur], acc0.at[dstb[cur]], s0sem[cur],
                             add=True),
            pltpu.async_copy(v1b[cur], acc1.at[dstb[cur]], s1sem[cur],
                             add=True),
        )
    for hs in sc_h.values():
        for h in hs:
            h.wait()
    plsc.subcore_barrier()
    sl = pl.ds(sid * PSC_NODES, PSC_NODES)
    pltpu.sync_copy(acc0.at[sl], out_hbm.at[cid, 0, sl])
    pltpu.sync_copy(acc1.at[sl], out_hbm.at[cid, 1, sl])


@functools.partial(
    pl.kernel,
    out_type=jax.ShapeDtypeStruct((NC, ACC_BINS), jnp.float32),
    mesh=_MESH,
    compiler_params=_SC_PARAMS,
    scratch_types=[
        pltpu.VMEM((PT_NODES,), jnp.float32),
        pltpu.VMEM((PT_NODES,), jnp.float32),
        pltpu.VMEM((PT_NODES,), jnp.float32),
        pltpu.VMEM((PT_NODES,), jnp.float32),
        pltpu.VMEM((PT_NODES,), jnp.float32),
        pltpu.VMEM((PT_NODES,), jnp.float32),
        pltpu.VMEM((PT_NODES,), jnp.int32),
        pltpu.VMEM((PT_NODES,), jnp.float32),
        pltpu.VMEM((PT_NODES,), jnp.float32),
        pltpu.VMEM((PT_NODES,), jnp.int32),
        pltpu.VMEM((PT_NODES,), jnp.int32),
        pltpu.VMEM((2, L), jnp.float32),
        pltpu.VMEM((ACC_BINS // NS,), jnp.float32),
        pltpu.SemaphoreType.DMA,
        pltpu.VMEM_SHARED((ACC_BINS,), jnp.float32),
    ],
)
def _pool_kernel(ap_hbm, z01_hbm, dinv_hbm, batch_hbm, b2_hbm,
                 out_hbm,
                 q00_v, q01_v, q10_v, q11_v, z01_v, dinv_v, batch_v,
                 h0_v, h1_v, ib_v, ic_v, b2_v, zb_v, sem, acc):
    cid = lax.axis_index("c")
    sid = lax.axis_index("s")
    wid = sid * NC + cid
    base = wid * PT_NODES
    sl = pl.ds(base, PT_NODES)
    hs = [pltpu.async_copy(ap_hbm.at[0, 0, sl], q00_v, sem),
          pltpu.async_copy(ap_hbm.at[1, 0, sl], q01_v, sem),
          pltpu.async_copy(ap_hbm.at[0, 1, sl], q10_v, sem),
          pltpu.async_copy(ap_hbm.at[1, 1, sl], q11_v, sem),
          pltpu.async_copy(z01_hbm.at[sl], z01_v, sem),
          pltpu.async_copy(dinv_hbm.at[sl], dinv_v, sem),
          pltpu.async_copy(batch_hbm.at[sl], batch_v, sem),
          pltpu.async_copy(b2_hbm, b2_v, sem)]
    for h in hs:
        h.wait()

    _fill(zb_v, ACC_BINS // NS, 0.0)
    _zero_shared_slice(acc, zb_v, ACC_BINS // NS, sid, ACC_BINS // NS)

    k64 = jnp.full((L,), 64, jnp.int32)
    k256 = jnp.full((L,), 256, jnp.int32)
    mhi = jnp.full((L,), MASK_HI, jnp.int32)

    def body(j, carry):
        s = pl.ds(j * L, L)
        dv = dinv_v[s]
        i = plsc.bitcast(z01_v[s], jnp.int32)
        z0 = plsc.bitcast(i & mhi, jnp.float32)
        z1 = plsc.bitcast(lax.shift_left(i, 16), jnp.float32)
        p = dv * (q00_v[s] + q01_v[s] + z0)
        q = dv * (q10_v[s] + q11_v[s] + z1)
        h0_v[s] = jnp.maximum(p + b2_v[0, :], 0.0)
        h1_v[s] = jnp.maximum(q + b2_v[1, :], 0.0)
        bi = batch_v[s]
        ib_v[s] = bi + k64
        ic_v[s] = bi + k256
        return carry

    lax.fori_loop(0, PT_NODES // L, body, 0)
    plsc.subcore_barrier()
    pltpu.sync_copy(h0_v, acc.at[batch_v], add=True)
    pltpu.sync_copy(h1_v, acc.at[ib_v], add=True)

    def body2(j, carry):
        h0_v[pl.ds(j * L, L)] = jnp.full((L,), 1.0, jnp.float32)
        return carry

    lax.fori_loop(0, PT_NODES // L, body2, 0)
    pltpu.sync_copy(h0_v, acc.at[ic_v], add=True)
    plsc.subcore_barrier()

    @pl.when(sid == 0)
    def _():
        pltpu.sync_copy(acc, out_hbm.at[cid])


def kernel(x, edge_index, batch, W1, b1, W2, b2):
    src = edge_index[0].astype(jnp.int32)
    dst = edge_index[1].astype(jnp.int32)
    xp = jnp.concatenate(
        [x[:, 0].astype(jnp.float32), jnp.zeros((NP - N_NODES,), jnp.float32)])
    bp = jnp.concatenate(
        [batch.astype(jnp.int32),
         jnp.full((NP - N_NODES,), PAD_GRAPH, jnp.int32)])
    W1f = W1.astype(jnp.float32)
    W2f = W2.astype(jnp.float32)
    w1t = jnp.tile(W1f.reshape(16, 1), (1, L))
    b1t = jnp.tile(b1.astype(jnp.float32).reshape(16, 1), (1, L))
    w20t = jnp.tile(W2f[:, 0].reshape(16, 1), (1, L))
    w21t = jnp.tile(W2f[:, 1].reshape(16, 1), (1, L))
    b2t = jnp.tile(b2.astype(jnp.float32).reshape(2, 1), (1, L))

    degp = _deg_kernel(dst)
    dinv2d, y2d = _dinv_y_tc(degp.reshape(2, NP // 128, 128),
                             xp.reshape(NP // 128, 128))
    dinv = dinv2d.reshape(NP)
    y = y2d.reshape(NP)
    agg1p = _edge_agg_kernel(src, dst, y)
    z01 = _feat_kernel(agg1p, dinv, y, w1t, b1t, w20t, w21t)
    ap = _edge_agg2_kernel(src, dst, z01)
    parts = _pool_kernel(ap, z01, dinv, bp, b2t)

    tot = parts[0] + parts[1]
    sums = jnp.stack([tot[0:NUM_GRAPHS], tot[64:64 + NUM_GRAPHS]], axis=1)
    cnt = tot[256:256 + NUM_GRAPHS]
    pooled = sums / jnp.clip(cnt, 1.0)[:, None]
    return jax.nn.log_softmax(pooled, axis=1)


# speedup vs baseline: 270.1579x; 1.0494x over previous
import functools

import jax
import jax.numpy as jnp
from jax import lax
from jax.experimental import pallas as pl
from jax.experimental.pallas import tpu as pltpu
from jax.experimental.pallas import tpu_sc as plsc

N_NODES = 100000
N_EDGES = 3200000
NUM_GRAPHS = 64

NC = 2
NS = 16
L = 16
NW = NC * NS

NP = 102400
PT_NODES = NP // NW
PSC_NODES = NP // NS
ET = N_EDGES // NW

TN = N_NODES
PAD_GRAPH = 512
ACC_BINS = 1024

_MESH = plsc.VectorSubcoreMesh(core_axis_name="c", subcore_axis_name="s")
_SC_PARAMS = pltpu.CompilerParams(needs_layout_passes=False)

MASK_HI = -65536


def _fill(ref, n, val):
    v = jnp.full((L,), val, ref.dtype)

    def body(j, carry):
        ref[pl.ds(j * L, L)] = v
        return carry

    lax.fori_loop(0, n // L, body, 0)


def _fill2(ref, n, val):
    v = jnp.full((2 * L,), val, ref.dtype)

    def body(j, carry):
        ref[pl.ds(j * 2 * L, 2 * L)] = v
        return carry

    lax.fori_loop(0, n // (2 * L), body, 0)


def _zero_shared_slice(acc, zbuf, zlen, sid, per_tile):
    base = sid * per_tile
    off = 0
    while off < per_tile:
        n = min(zlen, per_tile - off)
        pltpu.sync_copy(zbuf.at[pl.ds(0, n)], acc.at[pl.ds(base + off, n)])
        off += n


CD = 2000
NCD = ET // CD


@functools.partial(
    pl.kernel,
    out_type=jax.ShapeDtypeStruct((NC, NP), jnp.float32),
    mesh=_MESH,
    compiler_params=_SC_PARAMS,
    scratch_types=[
        pltpu.VMEM((CD,), jnp.int32),
        pltpu.VMEM((CD,), jnp.int32),
        pltpu.VMEM((CD,), jnp.int32),
        pltpu.VMEM((CD,), jnp.float32),
        pltpu.SemaphoreType.DMA,
        pltpu.SemaphoreType.DMA,
        pltpu.SemaphoreType.DMA,
        pltpu.SemaphoreType.DMA,
        pltpu.SemaphoreType.DMA,
        pltpu.SemaphoreType.DMA,
        pltpu.VMEM_SHARED((NP,), jnp.float32),
    ],
)
def _deg_kernel(dst_hbm, out_hbm, dst_a, dst_b, dst_c, ones_v,
                in_a, in_b, in_c, sc_a, sc_b, sc_c, acc):
    cid = lax.axis_index("c")
    sid = lax.axis_index("s")
    wid = sid * NC + cid
    dstb = [dst_a, dst_b, dst_c]
    insem = [in_a, in_b, in_c]
    scsem = [sc_a, sc_b, sc_c]
    _fill(ones_v, CD, 0.0)
    _zero_shared_slice(acc, ones_v, CD, sid, PSC_NODES)
    _fill(ones_v, CD, 1.0)
    plsc.subcore_barrier()
    ebase = wid * ET

    in_h = {}
    sc_h = {}
    for ci in range(min(2, NCD)):
        o = pl.ds(ebase + ci * CD, CD)
        in_h[ci] = pltpu.async_copy(dst_hbm.at[o], dstb[ci % 3], insem[ci % 3])
    for ci in range(NCD):
        cur = ci % 3
        in_h.pop(ci).wait()
        sc_h[ci] = pltpu.async_copy(ones_v, acc.at[dstb[cur]],
                                    scsem[cur], add=True)
        if ci + 2 < NCD:
            if ci - 1 >= 0:
                sc_h.pop(ci - 1).wait()
            o = pl.ds(ebase + (ci + 2) * CD, CD)
            in_h[ci + 2] = pltpu.async_copy(dst_hbm.at[o], dstb[(ci + 2) % 3],
                                            insem[(ci + 2) % 3])
    for h in sc_h.values():
        h.wait()
    plsc.subcore_barrier()
    sl = pl.ds(sid * PSC_NODES, PSC_NODES)
    pltpu.sync_copy(acc.at[sl], out_hbm.at[cid, sl])


def _dinv_y_tc_body(degp_ref, x_ref, dinv_ref, y_ref):
    deg = degp_ref[0] + degp_ref[1] + 1.0
    dinv = lax.rsqrt(deg)
    dinv_ref[...] = dinv
    y_ref[...] = x_ref[...] * dinv


_dinv_y_tc = pl.pallas_call(
    _dinv_y_tc_body,
    out_shape=(
        jax.ShapeDtypeStruct((NP // 128, 128), jnp.float32),
        jax.ShapeDtypeStruct((NP // 128, 128), jnp.float32),
    ),
)


CA = 2000
NCA = ET // CA
GUA = 5


@functools.partial(
    pl.kernel,
    out_type=jax.ShapeDtypeStruct((NC, NP), jnp.float32),
    mesh=_MESH,
    compiler_params=_SC_PARAMS,
    scratch_types=[
        pltpu.VMEM((TN,), jnp.float32),
        pltpu.VMEM((CA,), jnp.int32),
        pltpu.VMEM((CA,), jnp.int32),
        pltpu.VMEM((CA,), jnp.int32),
        pltpu.VMEM((CA,), jnp.int32),
        pltpu.VMEM((CA,), jnp.float32),
        pltpu.VMEM((CA,), jnp.float32),
        pltpu.SemaphoreType.DMA,
        pltpu.SemaphoreType.DMA,
        pltpu.SemaphoreType.DMA,
        pltpu.SemaphoreType.DMA,
        pltpu.SemaphoreType.DMA,
        pltpu.SemaphoreType.DMA,
        pltpu.VMEM_SHARED((NP,), jnp.float32),
    ],
)
def _edge_agg_kernel(src_hbm, dst_hbm, tab_hbm, out_hbm,
                     tab_v, src_a, src_b, dst_a, dst_b, vals_a, vals_b,
                     in_sa, in_sb, in_a, in_b, sc_a, sc_b, acc):
    cid = lax.axis_index("c")
    sid = lax.axis_index("s")
    wid = sid * NC + cid
    srcb = [src_a, src_b]
    dstb = [dst_a, dst_b]
    valb = [vals_a, vals_b]
    srcsem = [in_sa, in_sb]
    insem = [in_a, in_b]
    scsem = [sc_a, sc_b]
    pltpu.sync_copy(tab_hbm.at[pl.ds(0, TN)], tab_v)
    _fill(vals_a, CA, 0.0)
    _zero_shared_slice(acc, vals_a, CA, sid, PSC_NODES)
    plsc.subcore_barrier()
    ebase = wid * ET

    sc_h = {}
    o = pl.ds(ebase, CA)
    in_h = {0: (pltpu.async_copy(src_hbm.at[o], src_a, in_sa),
                pltpu.async_copy(dst_hbm.at[o], dst_a, in_a))}
    for ci in range(NCA):
        cur = ci % 2
        nxt = 1 - cur
        for h in in_h.pop(ci):
            h.wait()
        if ci - 2 in sc_h:
            sc_h.pop(ci - 2).wait()

        def g(j, c2, cur=cur):
            for f in range(GUA):
                s = pl.ds(j * (GUA * L) + f * L, L)
                valb[cur][s] = plsc.load_gather(tab_v, [srcb[cur][s]])
            return c2

        lax.fori_loop(0, CA // (GUA * L), g, 0)
        if ci + 1 < NCA:
            o = pl.ds(ebase + (ci + 1) * CA, CA)
            if ci - 1 in sc_h:
                sc_h.pop(ci - 1).wait()
            in_h[ci + 1] = (pltpu.async_copy(src_hbm.at[o], srcb[nxt],
                                             srcsem[nxt]),
                            pltpu.async_copy(dst_hbm.at[o], dstb[nxt],
                                             insem[nxt]))
        sc_h[ci] = pltpu.async_copy(valb[cur], acc.at[dstb[cur]],
                                    scsem[cur], add=True)
    for h in sc_h.values():
        h.wait()
    plsc.subcore_barrier()
    sl = pl.ds(sid * PSC_NODES, PSC_NODES)
    pltpu.sync_copy(acc.at[sl], out_hbm.at[cid, sl])


@functools.partial(
    pl.kernel,
    out_type=jax.ShapeDtypeStruct((NP,), jnp.float32),
    mesh=_MESH,
    compiler_params=_SC_PARAMS,
    scratch_types=[
        pltpu.VMEM((PT_NODES,), jnp.float32),
        pltpu.VMEM((PT_NODES,), jnp.float32),
        pltpu.VMEM((PT_NODES,), jnp.float32),
        pltpu.VMEM((PT_NODES,), jnp.float32),
        pltpu.VMEM((PT_NODES,), jnp.float32),
        pltpu.VMEM((16, L), jnp.float32),
        pltpu.VMEM((16, L), jnp.float32),
        pltpu.VMEM((16, L), jnp.float32),
        pltpu.VMEM((16, L), jnp.float32),
        pltpu.SemaphoreType.DMA,
    ],
)
def _feat_kernel(aggp_hbm, dinv_hbm, y_hbm, w1_hbm, b1_hbm, w20_hbm, w21_hbm,
                 z01_hbm,
                 p0_v, p1_v, dinv_v, y_v, z01_v, w1_v, b1_v, w20_v, w21_v,
                 sem):
    cid = lax.axis_index("c")
    sid = lax.axis_index("s")
    wid = sid * NC + cid
    base = wid * PT_NODES
    sl = pl.ds(base, PT_NODES)
    hs = [pltpu.async_copy(aggp_hbm.at[0, sl], p0_v, sem),
          pltpu.async_copy(aggp_hbm.at[1, sl], p1_v, sem),
          pltpu.async_copy(dinv_hbm.at[sl], dinv_v, sem),
          pltpu.async_copy(y_hbm.at[sl], y_v, sem),
          pltpu.async_copy(w1_hbm, w1_v, sem),
          pltpu.async_copy(b1_hbm, b1_v, sem),
          pltpu.async_copy(w20_hbm, w20_v, sem),
          pltpu.async_copy(w21_hbm, w21_v, sem)]
    for h in hs:
        h.wait()

    zero = jnp.zeros((L,), jnp.float32)
    mhi = jnp.full((L,), MASK_HI, jnp.int32)

    def body(j, carry):
        s = pl.ds(j * L, L)
        dv = dinv_v[s]
        s1 = dv * (p0_v[s] + p1_v[s] + y_v[s])
        g0 = zero
        g1 = zero
        for f in range(16):
            h = jnp.maximum(s1 * w1_v[f, :] + b1_v[f, :], 0.0)
            g0 = g0 + h * w20_v[f, :]
            g1 = g1 + h * w21_v[f, :]
        i0 = plsc.bitcast(g0 * dv, jnp.int32) & mhi
        i1 = lax.shift_right_logical(plsc.bitcast(g1 * dv, jnp.int32), 16)
        z01_v[s] = plsc.bitcast(i0 | i1, jnp.float32)
        return carry

    lax.fori_loop(0, PT_NODES // L, body, 0)
    pltpu.sync_copy(z01_v, z01_hbm.at[sl])


CB = 2000
NCB = ET // CB
GUB = 5


@functools.partial(
    pl.kernel,
    out_type=jax.ShapeDtypeStruct((NC, 2, NP), jnp.float32),
    mesh=_MESH,
    compiler_params=_SC_PARAMS,
    scratch_types=[
        pltpu.VMEM((TN,), jnp.float32),
        pltpu.VMEM((CB,), jnp.int32),
        pltpu.VMEM((CB,), jnp.int32),
        pltpu.VMEM((CB,), jnp.int32),
        pltpu.VMEM((CB,), jnp.int32),
        pltpu.VMEM((CB,), jnp.float32),
        pltpu.VMEM((CB,), jnp.float32),
        pltpu.VMEM((CB,), jnp.float32),
        pltpu.VMEM((CB,), jnp.float32),
        pltpu.SemaphoreType.DMA,
        pltpu.SemaphoreType.DMA,
        pltpu.SemaphoreType.DMA,
        pltpu.SemaphoreType.DMA,
        pltpu.SemaphoreType.DMA,
        pltpu.SemaphoreType.DMA,
        pltpu.SemaphoreType.DMA,
        pltpu.SemaphoreType.DMA,
        pltpu.VMEM_SHARED((NP,), jnp.float32),
        pltpu.VMEM_SHARED((NP,), jnp.float32),
    ],
)
def _edge_agg2_kernel(src_hbm, dst_hbm, tab_hbm, out_hbm,
                      tab_v, src_a, src_b, dst_a, dst_b, v0_a, v0_b, v1_a, v1_b,
                      in_sa, in_sb, in_a, in_b, s0_a, s0_b, s1_a, s1_b,
                      acc0, acc1):
    cid = lax.axis_index("c")
    sid = lax.axis_index("s")
    wid = sid * NC + cid
    srcb = [src_a, src_b]
    dstb = [dst_a, dst_b]
    v0b = [v0_a, v0_b]
    v1b = [v1_a, v1_b]
    srcsem = [in_sa, in_sb]
    insem = [in_a, in_b]
    s0sem = [s0_a, s0_b]
    s1sem = [s1_a, s1_b]
    pltpu.sync_copy(tab_hbm.at[pl.ds(0, TN)], tab_v)
    _fill(v0_a, CB, 0.0)
    _zero_shared_slice(acc0, v0_a, CB, sid, PSC_NODES)
    _zero_shared_slice(acc1, v0_a, CB, sid, PSC_NODES)
    plsc.subcore_barrier()
    ebase = wid * ET
    mhi = jnp.full((L,), MASK_HI, jnp.int32)

    sc_h = {}
    o = pl.ds(ebase, CB)
    in_h = {0: (pltpu.async_copy(src_hbm.at[o], src_a, in_sa),
                pltpu.async_copy(dst_hbm.at[o], dst_a, in_a))}
    for ci in range(NCB):
        cur = ci % 2
        nxt = 1 - cur
        for h in in_h.pop(ci):
            h.wait()
        if ci - 2 in sc_h:
            for h in sc_h.pop(ci - 2):
                h.wait()

        def g(j, c2, cur=cur):
            for f in range(GUB):
                s = pl.ds(j * (GUB * L) + f * L, L)
                i = plsc.bitcast(plsc.load_gather(tab_v, [srcb[cur][s]]),
                                 jnp.int32)
                v0b[cur][s] = plsc.bitcast(i & mhi, jnp.float32)
                v1b[cur][s] = plsc.bitcast(lax.shift_left(i, 16), jnp.float32)
            return c2

        lax.fori_loop(0, CB // (GUB * L), g, 0)
        if ci + 1 < NCB:
            o = pl.ds(ebase + (ci + 1) * CB, CB)
            if ci - 1 in sc_h:
                for h in sc_h.pop(ci - 1):
                    h.wait()
            in_h[ci + 1] = (pltpu.async_copy(src_hbm.at[o], srcb[nxt],
                                             srcsem[nxt]),
                            pltpu.async_copy(dst_hbm.at[o], dstb[nxt],
                                             insem[nxt]))
        sc_h[ci] = (
            pltpu.async_copy(v0b[cur], acc0.at[dstb[cur]], s0sem[cur],
                             add=True),
            pltpu.async_copy(v1b[cur], acc1.at[dstb[cur]], s1sem[cur],
                             add=True),
        )
    for hs in sc_h.values():
        for h in hs:
            h.wait()
    plsc.subcore_barrier()
    sl = pl.ds(sid * PSC_NODES, PSC_NODES)
    pltpu.sync_copy(acc0.at[sl], out_hbm.at[cid, 0, sl])
    pltpu.sync_copy(acc1.at[sl], out_hbm.at[cid, 1, sl])


@functools.partial(
    pl.kernel,
    out_type=jax.ShapeDtypeStruct((NC, ACC_BINS), jnp.float32),
    mesh=_MESH,
    compiler_params=_SC_PARAMS,
    scratch_types=[
        pltpu.VMEM((PT_NODES,), jnp.float32),
        pltpu.VMEM((PT_NODES,), jnp.float32),
        pltpu.VMEM((PT_NODES,), jnp.float32),
        pltpu.VMEM((PT_NODES,), jnp.float32),
        pltpu.VMEM((PT_NODES,), jnp.float32),
        pltpu.VMEM((PT_NODES,), jnp.float32),
        pltpu.VMEM((PT_NODES,), jnp.int32),
        pltpu.VMEM((PT_NODES,), jnp.float32),
        pltpu.VMEM((PT_NODES,), jnp.float32),
        pltpu.VMEM((PT_NODES,), jnp.int32),
        pltpu.VMEM((PT_NODES,), jnp.int32),
        pltpu.VMEM((2, L), jnp.float32),
        pltpu.VMEM((ACC_BINS // NS,), jnp.float32),
        pltpu.SemaphoreType.DMA,
        pltpu.VMEM_SHARED((ACC_BINS,), jnp.float32),
    ],
)
def _pool_kernel(ap_hbm, z01_hbm, dinv_hbm, batch_hbm, b2_hbm,
                 out_hbm,
                 q00_v, q01_v, q10_v, q11_v, z01_v, dinv_v, batch_v,
                 h0_v, h1_v, ib_v, ic_v, b2_v, zb_v, sem, acc):
    cid = lax.axis_index("c")
    sid = lax.axis_index("s")
    wid = sid * NC + cid
    base = wid * PT_NODES
    sl = pl.ds(base, PT_NODES)
    hs = [pltpu.async_copy(ap_hbm.at[0, 0, sl], q00_v, sem),
          pltpu.async_copy(ap_hbm.at[1, 0, sl], q01_v, sem),
          pltpu.async_copy(ap_hbm.at[0, 1, sl], q10_v, sem),
          pltpu.async_copy(ap_hbm.at[1, 1, sl], q11_v, sem),
          pltpu.async_copy(z01_hbm.at[sl], z01_v, sem),
          pltpu.async_copy(dinv_hbm.at[sl], dinv_v, sem),
          pltpu.async_copy(batch_hbm.at[sl], batch_v, sem),
          pltpu.async_copy(b2_hbm, b2_v, sem)]
    for h in hs:
        h.wait()

    _fill(zb_v, ACC_BINS // NS, 0.0)
    _zero_shared_slice(acc, zb_v, ACC_BINS // NS, sid, ACC_BINS // NS)

    k64 = jnp.full((L,), 64, jnp.int32)
    k256 = jnp.full((L,), 256, jnp.int32)
    mhi = jnp.full((L,), MASK_HI, jnp.int32)

    def body(j, carry):
        s = pl.ds(j * L, L)
        dv = dinv_v[s]
        i = plsc.bitcast(z01_v[s], jnp.int32)
        z0 = plsc.bitcast(i & mhi, jnp.float32)
        z1 = plsc.bitcast(lax.shift_left(i, 16), jnp.float32)
        p = dv * (q00_v[s] + q01_v[s] + z0)
        q = dv * (q10_v[s] + q11_v[s] + z1)
        h0_v[s] = jnp.maximum(p + b2_v[0, :], 0.0)
        h1_v[s] = jnp.maximum(q + b2_v[1, :], 0.0)
        bi = batch_v[s]
        ib_v[s] = bi + k64
        ic_v[s] = bi + k256
        return carry

    lax.fori_loop(0, PT_NODES // L, body, 0)
    plsc.subcore_barrier()
    pltpu.sync_copy(h0_v, acc.at[batch_v], add=True)
    pltpu.sync_copy(h1_v, acc.at[ib_v], add=True)

    def body2(j, carry):
        h0_v[pl.ds(j * L, L)] = jnp.full((L,), 1.0, jnp.float32)
        return carry

    lax.fori_loop(0, PT_NODES // L, body2, 0)
    pltpu.sync_copy(h0_v, acc.at[ic_v], add=True)
    plsc.subcore_barrier()

    @pl.when(sid == 0)
    def _():
        pltpu.sync_copy(acc, out_hbm.at[cid])


def kernel(x, edge_index, batch, W1, b1, W2, b2):
    src = edge_index[0].astype(jnp.int32)
    dst = edge_index[1].astype(jnp.int32)
    xp = jnp.concatenate(
        [x[:, 0].astype(jnp.float32), jnp.zeros((NP - N_NODES,), jnp.float32)])
    bp = jnp.concatenate(
        [batch.astype(jnp.int32),
         jnp.full((NP - N_NODES,), PAD_GRAPH, jnp.int32)])
    W1f = W1.astype(jnp.float32)
    W2f = W2.astype(jnp.float32)
    w1t = jnp.tile(W1f.reshape(16, 1), (1, L))
    b1t = jnp.tile(b1.astype(jnp.float32).reshape(16, 1), (1, L))
    w20t = jnp.tile(W2f[:, 0].reshape(16, 1), (1, L))
    w21t = jnp.tile(W2f[:, 1].reshape(16, 1), (1, L))
    b2t = jnp.tile(b2.astype(jnp.float32).reshape(2, 1), (1, L))

    degp = _deg_kernel(dst)
    dinv2d, y2d = _dinv_y_tc(degp.reshape(2, NP // 128, 128),
                             xp.reshape(NP // 128, 128))
    dinv = dinv2d.reshape(NP)
    y = y2d.reshape(NP)
    agg1p = _edge_agg_kernel(src, dst, y)
    z01 = _feat_kernel(agg1p, dinv, y, w1t, b1t, w20t, w21t)
    ap = _edge_agg2_kernel(src, dst, z01)
    parts = _pool_kernel(ap, z01, dinv, bp, b2t)

    tot = parts[0] + parts[1]
    sums = jnp.stack([tot[0:NUM_GRAPHS], tot[64:64 + NUM_GRAPHS]], axis=1)
    cnt = tot[256:256 + NUM_GRAPHS]
    pooled = sums / jnp.clip(cnt, 1.0)[:, None]
    return jax.nn.log_softmax(pooled, axis=1)
